# baseline (speedup 1.0000x reference)
import math

import jax
import jax.numpy as jnp
from jax import lax
from jax.experimental import pallas as pl
from jax.experimental.pallas import tpu as pltpu
from jax.experimental.pallas import tpu_sc as plsc

B, N, T = 16, 512, 3
HID, H, C = 128, 4, 32
E = 65536

_NC, _NS = 2, 16
_NW = _NC * _NS
_EPW = E // _NW
_SLICE = (N * N) // _NS


def _gelu(v):
    return 0.5 * v * (1.0 + jax.lax.erf(v * (1.0 / math.sqrt(2.0))))


def _sc_hist_kernel(src_ref, dst_ref, ones_ref, zeros_ref, out_ref,
                    shared, idx_v, ones_v):
    cid = lax.axis_index("c")
    sid = lax.axis_index("s")
    wid = cid * _NS + sid
    pltpu.sync_copy(zeros_ref.at[pl.ds(sid * _SLICE, _SLICE)],
                    shared.at[pl.ds(sid * _SLICE, _SLICE)])
    pltpu.sync_copy(src_ref.at[pl.ds(wid * _EPW, _EPW)], idx_v)
    pltpu.sync_copy(ones_ref, ones_v)
    s_loc = idx_v[...]
    pltpu.sync_copy(dst_ref.at[pl.ds(wid * _EPW, _EPW)], idx_v)
    idx_v[...] = idx_v[...] * N + s_loc
    plsc.subcore_barrier()
    pltpu.sync_copy(ones_v, shared.at[idx_v], add=True)
    plsc.subcore_barrier()
    pltpu.sync_copy(shared.at[pl.ds(sid * _SLICE, _SLICE)],
                    out_ref.at[cid, pl.ds(sid * _SLICE, _SLICE)])


def _edge_counts(edge_index):
    hist = pl.kernel(
        _sc_hist_kernel,
        out_type=jax.ShapeDtypeStruct((_NC, N * N), jnp.float32),
        mesh=plsc.VectorSubcoreMesh(core_axis_name="c", subcore_axis_name="s"),
        scratch_types=[
            pltpu.VMEM_SHARED((N * N,), jnp.float32),
            pltpu.VMEM((_EPW,), jnp.int32),
            pltpu.VMEM((_EPW,), jnp.float32),
        ],
    )
    return hist(edge_index[0], edge_index[1],
                jnp.ones((_EPW,), jnp.float32),
                jnp.zeros((N * N,), jnp.float32))


def _gat_layer(h, cntT, neg_maskT, Wl, bl, Wr, br, sgn_ref, bias,
               xlpT_scr, xrpT_scr, att_flat):
    xl = jnp.dot(h, Wl, preferred_element_type=jnp.float32) + bl
    xr = jnp.dot(h, Wr, preferred_element_type=jnp.float32) + br
    xlpT_scr[...] = (xl * att_flat).T
    xrpT_scr[...] = (xr * att_flat).T
    ones_row = jnp.ones((1, N), jnp.float32)
    ones_c = jnp.ones((1, C), jnp.float32)
    absmask = jnp.int32(0x7FFFFFFF)
    outs = []
    for hd in range(H):
        xl_h = xl[:, hd * C:(hd + 1) * C]
        al_row = jnp.dot(ones_c, xlpT_scr[hd * C:(hd + 1) * C, :],
                         preferred_element_type=jnp.float32)
        ar_row = jnp.dot(ones_c, xrpT_scr[hd * C:(hd + 1) * C, :],
                         preferred_element_type=jnp.float32)
        lin = jax.lax.dot_general(
            jnp.concatenate([1.5 * ar_row, ones_row], axis=0),
            jnp.concatenate([ones_row, 1.5 * al_row], axis=0),
            (((0,), (0,)), ((), ())),
            preferred_element_type=jnp.float32)

        def absterm(hc):
            vp = xrpT_scr[pl.ds(hc, 1), :]
            up = xlpT_scr[pl.ds(hc, 1), :]
            tp = jax.lax.dot_general(
                jnp.concatenate([vp, ones_row], axis=0),
                jnp.concatenate([ones_row, up], axis=0),
                (((0,), (0,)), ((), ())),
                preferred_element_type=jnp.float32)
            ti = (jax.lax.bitcast_convert_type(tp, jnp.int32) & absmask) \
                ^ sgn_ref[pl.ds(hc, 1), 0:1]
            return jax.lax.bitcast_convert_type(ti, jnp.float32)

        def body(c, accs):
            a1, a2 = accs
            return (a1 + absterm(hd * C + c),
                    a2 + absterm(hd * C + c + C // 2))

        m1, m2 = jax.lax.fori_loop(
            0, C // 2, body, (lin, jnp.zeros((N, N), jnp.float32)), unroll=8)
        m = m1 + m2
        alphaT = 0.4 * m
        am = alphaT + neg_maskT
        amax = jnp.max(am, axis=1, keepdims=True)
        amax = jnp.where(jnp.isfinite(amax), amax, 0.0)
        ex = jnp.exp(am - amax)
        w = cntT * ex
        den = jnp.sum(w, axis=1, keepdims=True)
        wn = w * (1.0 / (den + 1e-16))
        out_h = jnp.dot(wn, xl_h, preferred_element_type=jnp.float32)
        outs.append(out_h)
    return jnp.concatenate(outs, axis=1) + bias


def _main_kernel(xb_ref, cntT_ref,
                 W0t_ref, b0_ref, g0_ref, be0_ref,
                 Wl1_ref, bl1_ref, Wr1_ref, br1_ref, af1_ref, sgn1_ref,
                 bias1_ref,
                 Wl2_ref, bl2_ref, Wr2_ref, br2_ref, af2_ref, sgn2_ref,
                 bias2_ref,
                 Wout_ref, bout_ref, g1_ref, be1_ref,
                 out_ref, xlpT_scr, xrpT_scr):
    eps = 1e-5
    inv = 1.0 / math.sqrt(1.0 + eps)
    xb = xb_ref[0]
    cntT = cntT_ref[0] + cntT_ref[1]
    neg_maskT = jnp.where(cntT > 0.0, 0.0, -jnp.inf)

    z = jnp.dot(xb, W0t_ref[...], preferred_element_type=jnp.float32)
    z = z + b0_ref[...]
    z = g0_ref[...] * (z * inv) + be0_ref[...]
    h = _gelu(z)

    h = _gat_layer(h, cntT, neg_maskT, Wl1_ref[...], bl1_ref[...],
                   Wr1_ref[...], br1_ref[...], sgn1_ref,
                   bias1_ref[...], xlpT_scr, xrpT_scr, af1_ref[...])
    h = _gelu(h)
    h = _gat_layer(h, cntT, neg_maskT, Wl2_ref[...], bl2_ref[...],
                   Wr2_ref[...], br2_ref[...], sgn2_ref,
                   bias2_ref[...], xlpT_scr, xrpT_scr, af2_ref[...])
    h = _gelu(h)

    pooled = jnp.mean(h, axis=0, keepdims=True)
    o = jnp.dot(pooled, Wout_ref[...], preferred_element_type=jnp.float32)
    o = o + bout_ref[...]
    o = g1_ref[...] * (o * inv) + be1_ref[...]
    out_ref[0] = _gelu(o)


def kernel(x, edge_index, W0, b0, g0, be0, Wl1, bl1, Wr1, br1, att1, bias1,
           Wl2, bl2, Wr2, br2, att2, bias2, Wout, bout, g1, be1):
    cntT = _edge_counts(edge_index).reshape(_NC, N, N)

    xb = x.reshape(B, N, N * T)
    row = lambda v: v.reshape(1, HID)
    full = lambda a: pl.BlockSpec(a.shape, lambda b: (0,) * a.ndim)
    signbit = jnp.int32(-2147483648)

    def sgn_col(att):
        af = att.reshape(HID)
        return jnp.where(af < 0.0, signbit, 0).astype(jnp.int32).reshape(HID, 1)

    args = [xb, cntT,
            W0.T, row(b0), row(g0), row(be0),
            Wl1, row(bl1), Wr1, row(br1), att1.reshape(1, HID),
            sgn_col(att1), row(bias1),
            Wl2, row(bl2), Wr2, row(br2), att2.reshape(1, HID),
            sgn_col(att2), row(bias2),
            Wout, row(bout), row(g1), row(be1)]
    in_specs = [pl.BlockSpec((1, N, N * T), lambda b: (b, 0, 0))]
    in_specs += [full(a) for a in args[1:]]

    out = pl.pallas_call(
        _main_kernel,
        grid=(B,),
        in_specs=in_specs,
        out_specs=pl.BlockSpec((1, 1, HID), lambda b: (b, 0, 0)),
        out_shape=jax.ShapeDtypeStruct((B, 1, HID), jnp.float32),
        scratch_shapes=[
            pltpu.VMEM((HID, N), jnp.float32),
            pltpu.VMEM((HID, N), jnp.float32),
        ],
        compiler_params=pltpu.CompilerParams(
            dimension_semantics=("parallel",)),
    )(*args)
    return out.reshape(B, HID)


# speedup vs baseline: 1.0540x; 1.0540x over previous
import math

import jax
import jax.numpy as jnp
from jax import lax
from jax.experimental import pallas as pl
from jax.experimental.pallas import tpu as pltpu
from jax.experimental.pallas import tpu_sc as plsc

B, N, T = 16, 512, 3
HID, H, C = 128, 4, 32
E = 65536

_NC, _NS = 2, 16
_NW = _NC * _NS
_EPW = E // _NW
_SLICE = (N * N) // _NS


def _gelu(v):
    return 0.5 * v * (1.0 + jax.lax.erf(v * (1.0 / math.sqrt(2.0))))


def _sc_hist_kernel(src_ref, dst_ref, ones_ref, zeros_ref, out_ref,
                    shared, idx_v, ones_v):
    cid = lax.axis_index("c")
    sid = lax.axis_index("s")
    wid = cid * _NS + sid
    pltpu.sync_copy(zeros_ref.at[pl.ds(sid * _SLICE, _SLICE)],
                    shared.at[pl.ds(sid * _SLICE, _SLICE)])
    pltpu.sync_copy(src_ref.at[pl.ds(wid * _EPW, _EPW)], idx_v)
    pltpu.sync_copy(ones_ref, ones_v)
    s_loc = idx_v[...]
    pltpu.sync_copy(dst_ref.at[pl.ds(wid * _EPW, _EPW)], idx_v)
    idx_v[...] = idx_v[...] * N + s_loc
    plsc.subcore_barrier()
    pltpu.sync_copy(ones_v, shared.at[idx_v], add=True)
    plsc.subcore_barrier()
    pltpu.sync_copy(shared.at[pl.ds(sid * _SLICE, _SLICE)],
                    out_ref.at[cid, pl.ds(sid * _SLICE, _SLICE)])


def _edge_counts(edge_index):
    hist = pl.kernel(
        _sc_hist_kernel,
        out_type=jax.ShapeDtypeStruct((_NC, N * N), jnp.float32),
        mesh=plsc.VectorSubcoreMesh(core_axis_name="c", subcore_axis_name="s"),
        scratch_types=[
            pltpu.VMEM_SHARED((N * N,), jnp.float32),
            pltpu.VMEM((_EPW,), jnp.int32),
            pltpu.VMEM((_EPW,), jnp.float32),
        ],
    )
    return hist(edge_index[0], edge_index[1],
                jnp.ones((_EPW,), jnp.float32),
                jnp.zeros((N * N,), jnp.float32))


def _gat_layer(h, cntT, neg_maskT, Wl, bl, Wr, br, sgn_ref, bias,
               xlpT_scr, xrpT_scr, att_flat):
    xl = jnp.dot(h, Wl, preferred_element_type=jnp.float32) + bl
    xr = jnp.dot(h, Wr, preferred_element_type=jnp.float32) + br
    xlpT_scr[...] = (xl * att_flat).T
    xrpT_scr[...] = (xr * att_flat).T
    ones_row = jnp.ones((1, N), jnp.float32)
    ones_c = jnp.ones((1, C), jnp.float32)
    absmask = jnp.int32(0x7FFFFFFF)
    outs = []
    for hd in range(H):
        xl_h = xl[:, hd * C:(hd + 1) * C]
        al_row = jnp.dot(ones_c, xlpT_scr[hd * C:(hd + 1) * C, :],
                         preferred_element_type=jnp.float32)
        ar_row = jnp.dot(ones_c, xrpT_scr[hd * C:(hd + 1) * C, :],
                         preferred_element_type=jnp.float32)
        lin = jax.lax.dot_general(
            jnp.concatenate([1.5 * ar_row, ones_row], axis=0),
            jnp.concatenate([ones_row, 1.5 * al_row], axis=0),
            (((0,), (0,)), ((), ())),
            preferred_element_type=jnp.float32)

        def body(c, acc):
            vp = xrpT_scr[pl.ds(hd * C + c, 1), :]
            up = xlpT_scr[pl.ds(hd * C + c, 1), :]
            tp = jax.lax.dot_general(
                jnp.concatenate([vp, ones_row], axis=0),
                jnp.concatenate([ones_row, up], axis=0),
                (((0,), (0,)), ((), ())),
                preferred_element_type=jnp.float32)
            ti = (jax.lax.bitcast_convert_type(tp, jnp.int32) & absmask) \
                ^ sgn_ref[pl.ds(hd * C + c, 1), 0:1]
            return acc + jax.lax.bitcast_convert_type(ti, jnp.float32)

        m = jax.lax.fori_loop(0, C, body, lin, unroll=16)
        alphaT = 0.4 * m
        am = alphaT + neg_maskT
        amax = jnp.max(am, axis=1, keepdims=True)
        amax = jnp.where(jnp.isfinite(amax), amax, 0.0)
        ex = jnp.exp(am - amax)
        w = cntT * ex
        den = jnp.sum(w, axis=1, keepdims=True)
        wn = w * (1.0 / (den + 1e-16))
        out_h = jnp.dot(wn, xl_h, preferred_element_type=jnp.float32)
        outs.append(out_h)
    return jnp.concatenate(outs, axis=1) + bias


def _main_kernel(xb_ref, cntT_ref,
                 W0t_ref, b0_ref, g0_ref, be0_ref,
                 Wl1_ref, bl1_ref, Wr1_ref, br1_ref, af1_ref, sgn1_ref,
                 bias1_ref,
                 Wl2_ref, bl2_ref, Wr2_ref, br2_ref, af2_ref, sgn2_ref,
                 bias2_ref,
                 Wout_ref, bout_ref, g1_ref, be1_ref,
                 out_ref, xlpT_scr, xrpT_scr):
    eps = 1e-5
    inv = 1.0 / math.sqrt(1.0 + eps)
    xb = xb_ref[0]
    cntT = cntT_ref[0] + cntT_ref[1]
    neg_maskT = jnp.where(cntT > 0.0, 0.0, -jnp.inf)

    z = jnp.dot(xb, W0t_ref[...], preferred_element_type=jnp.float32)
    z = z + b0_ref[...]
    z = g0_ref[...] * (z * inv) + be0_ref[...]
    h = _gelu(z)

    h = _gat_layer(h, cntT, neg_maskT, Wl1_ref[...], bl1_ref[...],
                   Wr1_ref[...], br1_ref[...], sgn1_ref,
                   bias1_ref[...], xlpT_scr, xrpT_scr, af1_ref[...])
    h = _gelu(h)
    h = _gat_layer(h, cntT, neg_maskT, Wl2_ref[...], bl2_ref[...],
                   Wr2_ref[...], br2_ref[...], sgn2_ref,
                   bias2_ref[...], xlpT_scr, xrpT_scr, af2_ref[...])
    h = _gelu(h)

    pooled = jnp.mean(h, axis=0, keepdims=True)
    o = jnp.dot(pooled, Wout_ref[...], preferred_element_type=jnp.float32)
    o = o + bout_ref[...]
    o = g1_ref[...] * (o * inv) + be1_ref[...]
    out_ref[0] = _gelu(o)


def kernel(x, edge_index, W0, b0, g0, be0, Wl1, bl1, Wr1, br1, att1, bias1,
           Wl2, bl2, Wr2, br2, att2, bias2, Wout, bout, g1, be1):
    cntT = _edge_counts(edge_index).reshape(_NC, N, N)

    xb = x.reshape(B, N, N * T)
    row = lambda v: v.reshape(1, HID)
    full = lambda a: pl.BlockSpec(a.shape, lambda b: (0,) * a.ndim)
    signbit = jnp.int32(-2147483648)

    def sgn_col(att):
        af = att.reshape(HID)
        return jnp.where(af < 0.0, signbit, 0).astype(jnp.int32).reshape(HID, 1)

    args = [xb, cntT,
            W0.T, row(b0), row(g0), row(be0),
            Wl1, row(bl1), Wr1, row(br1), att1.reshape(1, HID),
            sgn_col(att1), row(bias1),
            Wl2, row(bl2), Wr2, row(br2), att2.reshape(1, HID),
            sgn_col(att2), row(bias2),
            Wout, row(bout), row(g1), row(be1)]
    in_specs = [pl.BlockSpec((1, N, N * T), lambda b: (b, 0, 0))]
    in_specs += [full(a) for a in args[1:]]

    out = pl.pallas_call(
        _main_kernel,
        grid=(B,),
        in_specs=in_specs,
        out_specs=pl.BlockSpec((1, 1, HID), lambda b: (b, 0, 0)),
        out_shape=jax.ShapeDtypeStruct((B, 1, HID), jnp.float32),
        scratch_shapes=[
            pltpu.VMEM((HID, N), jnp.float32),
            pltpu.VMEM((HID, N), jnp.float32),
        ],
        compiler_params=pltpu.CompilerParams(
            dimension_semantics=("parallel",)),
    )(*args)
    return out.reshape(B, HID)
